# Initial kernel scaffold; baseline (speedup 1.0000x reference)
import jax
import jax.numpy as jnp
from jax.experimental import pallas as pl


def kernel(hidden_states, router_weight, router_bias, gate_w, up_w, down_w, shared_gate_w, shared_up_w, shared_down_w):
    raise NotImplementedError("write your pallas kernel here")


# speedup vs baseline: 1.0602x; 1.0602x over previous
import jax
import jax.numpy as jnp
from jax import lax
from jax.experimental import pallas as pl
from jax.experimental.pallas import tpu as pltpu

B, S, H = 2, 1024, 1024
I = 512
E = 32
TOP_K = 4
N_GROUP = 4
SCALE = 2.5
SH_I = 1024
T = B * S

_TB = 256
_M = 256
_NT = 64
_ST = _M + 1
_P = H // 128


def _routing_kernel(x_ref, rw_ref, rb_ref, idx_ref, w_ref):
    xb = x_ref[...]
    logits = lax.dot_general(xb, rw_ref[...], (((1,), (1,)), ((), ())),
                             preferred_element_type=jnp.float32)
    scores = jax.nn.sigmoid(logits)
    sfc = scores + rb_ref[...]
    iota = lax.broadcasted_iota(jnp.int32, (_TB, E), 1)
    gid = iota // (E // N_GROUP)
    gs_cols = []
    for g in range(N_GROUP):
        vals = jnp.where(gid == g, sfc, -1e30)
        m1 = jnp.max(vals, axis=1, keepdims=True)
        a1 = jnp.min(jnp.where(vals == m1, iota, E), axis=1, keepdims=True)
        m2 = jnp.max(jnp.where(iota == a1, -1e30, vals), axis=1, keepdims=True)
        gs_cols.append(m1 + m2)
    gs = jnp.concatenate(gs_cols, axis=1)
    iota_g = lax.broadcasted_iota(jnp.int32, (_TB, N_GROUP), 1)
    m1 = jnp.max(gs, axis=1, keepdims=True)
    g1 = jnp.min(jnp.where(gs == m1, iota_g, N_GROUP), axis=1, keepdims=True)
    gs2 = jnp.where(iota_g == g1, -1e30, gs)
    m2 = jnp.max(gs2, axis=1, keepdims=True)
    g2 = jnp.min(jnp.where(gs2 == m2, iota_g, N_GROUP), axis=1, keepdims=True)
    tmp = jnp.where((gid == g1) | (gid == g2), sfc, 0.0)
    icols, wcols = [], []
    for _ in range(TOP_K):
        mk = jnp.max(tmp, axis=1, keepdims=True)
        ik = jnp.min(jnp.where(tmp == mk, iota, E), axis=1, keepdims=True)
        sel = iota == ik
        wcols.append(jnp.sum(jnp.where(sel, scores, 0.0), axis=1,
                             keepdims=True))
        tmp = jnp.where(sel, -1.0, tmp)
        icols.append(ik)
    ti = jnp.concatenate(icols, axis=1)
    tw = jnp.concatenate(wcols, axis=1)
    tw = tw / (jnp.sum(tw, axis=1, keepdims=True) + 1e-20) * SCALE
    idx_ref[...] = ti
    w_ref[...] = tw


def _expert_kernel(te_ref, rt_ref, rw_ref, xs_ref, gw_ref, uw_ref, dw_ref,
                   out_ref, xt_ref, yt_ref):
    c = pl.program_id(0)
    j = pl.program_id(1)
    tile = c * (_NT // 2) + j

    @pl.when(j == 0)
    def _():
        out_ref[...] = jnp.zeros_like(out_ref)

    for mi in range(_M):
        src = pl.multiple_of(rt_ref[tile, mi], _P)
        xt_ref[mi:mi + _P * _ST:_ST, :] = xs_ref[pl.ds(src, _P), :]
    x = jnp.concatenate(
        [xt_ref[pl.ds(k * _ST, _M), :] for k in range(_P)], axis=-1)
    g = lax.dot_general(x, gw_ref[0], (((1,), (1,)), ((), ())),
                        preferred_element_type=jnp.float32)
    u = lax.dot_general(x, uw_ref[0], (((1,), (1,)), ((), ())),
                        preferred_element_type=jnp.float32)
    h = g * jax.nn.sigmoid(g) * u
    y = lax.dot_general(h, dw_ref[0], (((1,), (1,)), ((), ())),
                        preferred_element_type=jnp.float32)
    for k in range(_P):
        yt_ref[pl.ds(k * _ST, _M), :] = y[:, k * 128:(k + 1) * 128]
    unroll = 4
    for base in range(0, _M, unroll):
        updated = []
        for q in range(unroll):
            mi = base + q
            dst = pl.multiple_of(rt_ref[tile, mi], _P)
            w = rw_ref[tile, mi]
            updated.append(out_ref[0, pl.ds(dst, _P), :]
                           + w * yt_ref[mi:mi + _P * _ST:_ST, :])
        for q in range(unroll):
            mi = base + q
            dst = pl.multiple_of(rt_ref[tile, mi], _P)
            out_ref[0, pl.ds(dst, _P), :] = updated[q]


def _combine_kernel(x_ref, sg_ref, su_ref, sd_ref, p_ref, o_ref):
    xb = x_ref[...]
    g = lax.dot_general(xb, sg_ref[...], (((1,), (1,)), ((), ())),
                        preferred_element_type=jnp.float32)
    u = lax.dot_general(xb, su_ref[...], (((1,), (1,)), ((), ())),
                        preferred_element_type=jnp.float32)
    h = g * jax.nn.sigmoid(g) * u
    sh = lax.dot_general(h, sd_ref[...], (((1,), (1,)), ((), ())),
                         preferred_element_type=jnp.float32)
    o_ref[...] = sh + p_ref[0] + p_ref[1]


def kernel(hidden_states, router_weight, router_bias, gate_w, up_w, down_w,
           shared_gate_w, shared_up_w, shared_down_w):
    tokens = hidden_states.reshape(T, H)
    nb = T // _TB // 2

    ti, tw = pl.pallas_call(
        _routing_kernel,
        grid=(2, nb),
        in_specs=[
            pl.BlockSpec((_TB, H), lambda c, b: (c * nb + b, 0)),
            pl.BlockSpec((E, H), lambda c, b: (0, 0)),
            pl.BlockSpec((1, E), lambda c, b: (0, 0)),
        ],
        out_specs=[
            pl.BlockSpec((_TB, TOP_K), lambda c, b: (c * nb + b, 0)),
            pl.BlockSpec((_TB, TOP_K), lambda c, b: (c * nb + b, 0)),
        ],
        out_shape=[
            jax.ShapeDtypeStruct((T, TOP_K), jnp.int32),
            jax.ShapeDtypeStruct((T, TOP_K), jnp.float32),
        ],
        compiler_params=pltpu.CompilerParams(
            dimension_semantics=("parallel", "arbitrary")),
        name="moe_routing",
    )(tokens, router_weight, router_bias.reshape(1, E))

    flat_e = ti.reshape(-1)
    flat_w = tw.reshape(-1)
    perm = jnp.argsort(flat_e, stable=True).astype(jnp.int32)
    counts = jnp.sum(
        (flat_e[:, None] == jnp.arange(E, dtype=jnp.int32)[None, :])
        .astype(jnp.int32), axis=0)
    padded = ((counts + _M - 1) // _M) * _M
    pend = jnp.cumsum(padded)
    pstart = pend - padded
    offsets = jnp.cumsum(counts) - counts
    tile_start = jnp.arange(_NT, dtype=jnp.int32) * _M
    tile_expert = jnp.minimum(
        jnp.sum((tile_start[:, None] >= pend[None, :]).astype(jnp.int32),
                axis=1), E - 1).astype(jnp.int32)
    pos = jnp.arange(_NT * _M, dtype=jnp.int32)
    e_p = jnp.minimum(
        jnp.sum((pos[:, None] >= pend[None, :]).astype(jnp.int32), axis=1),
        E - 1).astype(jnp.int32)
    r = pos - jnp.take(pstart, e_p)
    valid = r < jnp.take(counts, e_p)
    srank = jnp.clip(jnp.take(offsets, e_p) + r, 0, T * TOP_K - 1)
    flat_i = jnp.take(perm, srank)
    row_token = jnp.where(valid, flat_i // TOP_K, 0).astype(jnp.int32)
    row_weight = jnp.where(valid, jnp.take(flat_w, flat_i), 0.0)
    rt8 = (row_token * _P).reshape(_NT, _M)
    rw_tab = row_weight.astype(jnp.float32).reshape(_NT, _M)

    xs = tokens.reshape(T * _P, 128)
    nt2 = _NT // 2
    partial = pl.pallas_call(
        _expert_kernel,
        grid_spec=pltpu.PrefetchScalarGridSpec(
            num_scalar_prefetch=3,
            grid=(2, nt2),
            in_specs=[
                pl.BlockSpec((T * _P, 128), lambda c, j, te, rt, rw: (0, 0)),
                pl.BlockSpec((1, I, H),
                             lambda c, j, te, rt, rw: (te[c * nt2 + j], 0, 0)),
                pl.BlockSpec((1, I, H),
                             lambda c, j, te, rt, rw: (te[c * nt2 + j], 0, 0)),
                pl.BlockSpec((1, H, I),
                             lambda c, j, te, rt, rw: (te[c * nt2 + j], 0, 0)),
            ],
            out_specs=pl.BlockSpec((1, T * _P, 128),
                                   lambda c, j, te, rt, rw: (c, 0, 0)),
            scratch_shapes=[
                pltpu.VMEM((_ST * _P, 128), jnp.float32),
                pltpu.VMEM((_ST * _P, 128), jnp.float32),
            ],
        ),
        out_shape=jax.ShapeDtypeStruct((2, T * _P, 128), jnp.float32),
        compiler_params=pltpu.CompilerParams(
            dimension_semantics=("parallel", "arbitrary"),
            vmem_limit_bytes=50 * 1024 * 1024),
        name="moe_experts",
    )(tile_expert, rt8, rw_tab, xs, gate_w, up_w, down_w)

    p2 = partial.reshape(2, T, H)
    out = pl.pallas_call(
        _combine_kernel,
        grid=(2, nb),
        in_specs=[
            pl.BlockSpec((_TB, H), lambda c, b: (c * nb + b, 0)),
            pl.BlockSpec((SH_I, H), lambda c, b: (0, 0)),
            pl.BlockSpec((SH_I, H), lambda c, b: (0, 0)),
            pl.BlockSpec((H, SH_I), lambda c, b: (0, 0)),
            pl.BlockSpec((2, _TB, H), lambda c, b: (0, c * nb + b, 0)),
        ],
        out_specs=pl.BlockSpec((_TB, H), lambda c, b: (c * nb + b, 0)),
        out_shape=jax.ShapeDtypeStruct((T, H), jnp.float32),
        compiler_params=pltpu.CompilerParams(
            dimension_semantics=("parallel", "arbitrary"),
            vmem_limit_bytes=50 * 1024 * 1024),
        name="moe_shared_combine",
    )(tokens, shared_gate_w, shared_up_w, shared_down_w, p2)
    return out.reshape(B, S, H)


# speedup vs baseline: 2.6603x; 2.5092x over previous
import jax
import jax.numpy as jnp
from jax import lax
from jax.experimental import pallas as pl
from jax.experimental.pallas import tpu as pltpu

B, S, H = 2, 1024, 1024
I = 512
E = 32
TOP_K = 4
N_GROUP = 4
SCALE = 2.5
SH_I = 1024
T = B * S

_TB = 256
_M = 256
_NT = 64
_ST = _M + 1
_P = H // 128


def _routing_kernel(x_ref, rw_ref, rb_ref, idx_ref, w_ref):
    lg = lax.dot_general(rw_ref[...], x_ref[...], (((1,), (1,)), ((), ())),
                         preferred_element_type=jnp.float32)
    scores = jax.nn.sigmoid(lg)
    sfc = scores + rb_ref[...]
    iota = lax.broadcasted_iota(jnp.int32, (E, _TB), 0)
    gsz = E // N_GROUP
    gs_rows = []
    for g in range(N_GROUP):
        v = sfc[g * gsz:(g + 1) * gsz]
        io = iota[g * gsz:(g + 1) * gsz]
        m1 = jnp.max(v, axis=0, keepdims=True)
        a1 = jnp.min(jnp.where(v == m1, io, E), axis=0, keepdims=True)
        m2 = jnp.max(jnp.where(io == a1, -1e30, v), axis=0, keepdims=True)
        gs_rows.append(m1 + m2)
    gsc = jnp.concatenate(gs_rows, axis=0)
    iog = lax.broadcasted_iota(jnp.int32, (N_GROUP, _TB), 0)
    m1 = jnp.max(gsc, axis=0, keepdims=True)
    g1 = jnp.min(jnp.where(gsc == m1, iog, N_GROUP), axis=0, keepdims=True)
    gs2 = jnp.where(iog == g1, -1e30, gsc)
    m2 = jnp.max(gs2, axis=0, keepdims=True)
    g2 = jnp.min(jnp.where(gs2 == m2, iog, N_GROUP), axis=0, keepdims=True)
    gid = iota // gsz
    tmp = jnp.where((gid == g1) | (gid == g2), sfc, 0.0)
    irows, wrows = [], []
    for _ in range(TOP_K):
        mk = jnp.max(tmp, axis=0, keepdims=True)
        ik = jnp.min(jnp.where(tmp == mk, iota, E), axis=0, keepdims=True)
        sel = iota == ik
        wrows.append(jnp.sum(jnp.where(sel, scores, 0.0), axis=0,
                             keepdims=True))
        tmp = jnp.where(sel, -1.0, tmp)
        irows.append(ik)
    ti = jnp.concatenate(irows, axis=0)
    tw = jnp.concatenate(wrows, axis=0)
    tw = tw / (jnp.sum(tw, axis=0, keepdims=True) + 1e-20) * SCALE
    idx_ref[...] = ti
    w_ref[...] = tw


def _expert_kernel(rt_ref, rw_ref, nck_ref, bt_ref, xs_ref, gw_ref, uw_ref,
                   dw_ref, out_ref, xt_ref, yt_ref):
    e = pl.program_id(0)
    n = nck_ref[e]

    @pl.when(e == 0)
    def _():
        out_ref[...] = jnp.zeros_like(out_ref)

    def _gather(tile, xt):
        for mi in range(_M):
            src = pl.multiple_of(rt_ref[tile, mi], _P)
            xt[mi:mi + _P * _ST:_ST, :] = xs_ref[pl.ds(src, _P), :]

    @pl.when(n > 0)
    def _():
        _gather(bt_ref[e], xt_ref.at[0])

    def _chunk(ci, carry):
        tile = bt_ref[e] + ci
        cur = xt_ref.at[lax.rem(ci, 2)]
        x = jnp.concatenate(
            [cur[pl.ds(k * _ST, _M), :] for k in range(_P)], axis=-1)
        g = lax.dot_general(x, gw_ref[0], (((1,), (1,)), ((), ())),
                            preferred_element_type=jnp.float32)
        u = lax.dot_general(x, uw_ref[0], (((1,), (1,)), ((), ())),
                            preferred_element_type=jnp.float32)
        h = g * jax.nn.sigmoid(g) * u
        y = lax.dot_general(h, dw_ref[0], (((1,), (1,)), ((), ())),
                            preferred_element_type=jnp.float32)

        @pl.when(ci + 1 < n)
        def _():
            _gather(tile + 1, xt_ref.at[lax.rem(ci + 1, 2)])

        for k in range(_P):
            yt_ref[pl.ds(k * _ST, _M), :] = y[:, k * 128:(k + 1) * 128]
        unroll = 4
        for base in range(0, _M, unroll):
            updated = []
            for q in range(unroll):
                mi = base + q
                dst = pl.multiple_of(rt_ref[tile, mi], _P)
                w = rw_ref[tile, mi]
                updated.append(out_ref[pl.ds(dst, _P), :]
                               + w * yt_ref[mi:mi + _P * _ST:_ST, :])
            for q in range(unroll):
                mi = base + q
                dst = pl.multiple_of(rt_ref[tile, mi], _P)
                out_ref[pl.ds(dst, _P), :] = updated[q]
        return carry

    lax.fori_loop(0, n, _chunk, 0)


def _scatter_kernel(dst_ref, fw_ref, ps_ref, pe_ref, ot_ref, ow_ref):
    def put(o, carry):
        for q in range(8):
            i = o * 8 + q
            d = dst_ref[i]
            ot_ref[d] = (i & (T - 1)) * _P
            ow_ref[d] = fw_ref[i]
        return carry

    lax.fori_loop(0, T * TOP_K // 8, put, 0)

    def pad_expert(e, carry):
        def fill(p, c2):
            ot_ref[p] = 0
            ow_ref[p] = 0.0
            return c2
        return lax.fori_loop(ps_ref[e], pe_ref[e], fill, carry)

    lax.fori_loop(0, E, pad_expert, 0)


def _combine_kernel(x_ref, sg_ref, su_ref, sd_ref, p_ref, o_ref):
    xb = x_ref[...]
    g = lax.dot_general(xb, sg_ref[...], (((1,), (1,)), ((), ())),
                        preferred_element_type=jnp.float32)
    u = lax.dot_general(xb, su_ref[...], (((1,), (1,)), ((), ())),
                        preferred_element_type=jnp.float32)
    h = g * jax.nn.sigmoid(g) * u
    sh = lax.dot_general(h, sd_ref[...], (((1,), (1,)), ((), ())),
                         preferred_element_type=jnp.float32)
    o_ref[...] = sh + p_ref[...]


def kernel(hidden_states, router_weight, router_bias, gate_w, up_w, down_w,
           shared_gate_w, shared_up_w, shared_down_w):
    tokens = hidden_states.reshape(T, H)
    nb = T // _TB // 2

    ti, tw = pl.pallas_call(
        _routing_kernel,
        grid=(T // _TB,),
        in_specs=[
            pl.BlockSpec((_TB, H), lambda b: (b, 0)),
            pl.BlockSpec((E, H), lambda b: (0, 0)),
            pl.BlockSpec((E, 1), lambda b: (0, 0)),
        ],
        out_specs=[
            pl.BlockSpec((TOP_K, _TB), lambda b: (0, b)),
            pl.BlockSpec((TOP_K, _TB), lambda b: (0, b)),
        ],
        out_shape=[
            jax.ShapeDtypeStruct((TOP_K, T), jnp.int32),
            jax.ShapeDtypeStruct((TOP_K, T), jnp.float32),
        ],
        compiler_params=pltpu.CompilerParams(
            dimension_semantics=("arbitrary",)),
        name="moe_routing",
    )(tokens, router_weight, router_bias.reshape(E, 1))

    flat_e = ti.reshape(-1)
    flat_w = tw.reshape(-1)
    nblk, blk = 64, T * TOP_K // 64
    oh = (flat_e[:, None] == jnp.arange(E, dtype=jnp.int32)[None, :]
          ).astype(jnp.float32)
    ohb = oh.reshape(nblk, blk, E)
    tril_b = jnp.tril(jnp.ones((blk, blk), jnp.float32))
    within = jnp.einsum('ij,bjk->bik', tril_b, ohb)
    blkc = jnp.sum(ohb, axis=1)
    tril_n = jnp.tril(jnp.ones((nblk, nblk), jnp.float32), k=-1)
    base = tril_n @ blkc
    counts = jnp.sum(blkc, axis=0).astype(jnp.int32)
    padded = ((counts + _M - 1) // _M) * _M
    pend = jnp.cumsum(padded)
    pstart = pend - padded
    nck = (padded // _M).astype(jnp.int32)
    btile = (pstart // _M).astype(jnp.int32)
    slot = within + (base[:, None, :] + pstart[None, None, :].astype(
        jnp.float32))
    dst = (jnp.sum(slot.reshape(T * TOP_K, E) * oh, axis=1)
           ).astype(jnp.int32) - 1
    rt_flat, rw_flat = pl.pallas_call(
        _scatter_kernel,
        in_specs=[
            pl.BlockSpec(memory_space=pltpu.SMEM),
            pl.BlockSpec(memory_space=pltpu.SMEM),
            pl.BlockSpec(memory_space=pltpu.SMEM),
            pl.BlockSpec(memory_space=pltpu.SMEM),
        ],
        out_specs=[
            pl.BlockSpec(memory_space=pltpu.SMEM),
            pl.BlockSpec(memory_space=pltpu.SMEM),
        ],
        out_shape=[
            jax.ShapeDtypeStruct((_NT * _M,), jnp.int32),
            jax.ShapeDtypeStruct((_NT * _M,), jnp.float32),
        ],
        name="moe_table_scatter",
    )(dst, flat_w, pstart + counts, pend)
    rt8 = rt_flat.reshape(_NT, _M)
    rw_tab = rw_flat.reshape(_NT, _M)

    xs = tokens.reshape(T * _P, 128)
    partial = pl.pallas_call(
        _expert_kernel,
        grid_spec=pltpu.PrefetchScalarGridSpec(
            num_scalar_prefetch=4,
            grid=(E,),
            in_specs=[
                pl.BlockSpec((T * _P, 128), lambda e, rt, rw, nc, bt: (0, 0)),
                pl.BlockSpec((1, I, H), lambda e, rt, rw, nc, bt: (e, 0, 0)),
                pl.BlockSpec((1, I, H), lambda e, rt, rw, nc, bt: (e, 0, 0)),
                pl.BlockSpec((1, H, I), lambda e, rt, rw, nc, bt: (e, 0, 0)),
            ],
            out_specs=pl.BlockSpec((T * _P, 128),
                                   lambda e, rt, rw, nc, bt: (0, 0)),
            scratch_shapes=[
                pltpu.VMEM((2, _ST * _P, 128), jnp.float32),
                pltpu.VMEM((_ST * _P, 128), jnp.float32),
            ],
        ),
        out_shape=jax.ShapeDtypeStruct((T * _P, 128), jnp.float32),
        compiler_params=pltpu.CompilerParams(
            dimension_semantics=("arbitrary",),
            vmem_limit_bytes=50 * 1024 * 1024),
        name="moe_experts",
    )(rt8, rw_tab, nck, btile, xs, gate_w, up_w, down_w)

    p2 = partial.reshape(T, H)
    out = pl.pallas_call(
        _combine_kernel,
        grid=(T // _TB,),
        in_specs=[
            pl.BlockSpec((_TB, H), lambda b: (b, 0)),
            pl.BlockSpec((SH_I, H), lambda b: (0, 0)),
            pl.BlockSpec((SH_I, H), lambda b: (0, 0)),
            pl.BlockSpec((H, SH_I), lambda b: (0, 0)),
            pl.BlockSpec((_TB, H), lambda b: (b, 0)),
        ],
        out_specs=pl.BlockSpec((_TB, H), lambda b: (b, 0)),
        out_shape=jax.ShapeDtypeStruct((T, H), jnp.float32),
        compiler_params=pltpu.CompilerParams(
            dimension_semantics=("arbitrary",),
            vmem_limit_bytes=50 * 1024 * 1024),
        name="moe_shared_combine",
    )(tokens, shared_gate_w, shared_up_w, shared_down_w, p2)
    return out.reshape(B, S, H)


# speedup vs baseline: 2.7482x; 1.0330x over previous
import jax
import jax.numpy as jnp
from jax import lax
from jax.experimental import pallas as pl
from jax.experimental.pallas import tpu as pltpu

B, S, H = 2, 1024, 1024
I = 512
E = 32
TOP_K = 4
N_GROUP = 4
SCALE = 2.5
SH_I = 1024
T = B * S

_TB = 256
_M = 256
_NT = 64
_ST = _M + 1
_P = H // 128


def _routing_kernel(x_ref, rw_ref, rb_ref, idx_ref, w_ref):
    lg = lax.dot_general(rw_ref[...], x_ref[...], (((1,), (1,)), ((), ())),
                         preferred_element_type=jnp.float32)
    scores = jax.nn.sigmoid(lg)
    sfc = scores + rb_ref[...]
    iota = lax.broadcasted_iota(jnp.int32, (E, _TB), 0)
    gsz = E // N_GROUP
    gs_rows = []
    for g in range(N_GROUP):
        v = sfc[g * gsz:(g + 1) * gsz]
        io = iota[g * gsz:(g + 1) * gsz]
        m1 = jnp.max(v, axis=0, keepdims=True)
        a1 = jnp.min(jnp.where(v == m1, io, E), axis=0, keepdims=True)
        m2 = jnp.max(jnp.where(io == a1, -1e30, v), axis=0, keepdims=True)
        gs_rows.append(m1 + m2)
    gsc = jnp.concatenate(gs_rows, axis=0)
    iog = lax.broadcasted_iota(jnp.int32, (N_GROUP, _TB), 0)
    m1 = jnp.max(gsc, axis=0, keepdims=True)
    g1 = jnp.min(jnp.where(gsc == m1, iog, N_GROUP), axis=0, keepdims=True)
    gs2 = jnp.where(iog == g1, -1e30, gsc)
    m2 = jnp.max(gs2, axis=0, keepdims=True)
    g2 = jnp.min(jnp.where(gs2 == m2, iog, N_GROUP), axis=0, keepdims=True)
    gid = iota // gsz
    tmp = jnp.where((gid == g1) | (gid == g2), sfc, 0.0)
    irows, wrows = [], []
    for _ in range(TOP_K):
        mk = jnp.max(tmp, axis=0, keepdims=True)
        ik = jnp.min(jnp.where(tmp == mk, iota, E), axis=0, keepdims=True)
        sel = iota == ik
        wrows.append(jnp.sum(jnp.where(sel, scores, 0.0), axis=0,
                             keepdims=True))
        tmp = jnp.where(sel, -1.0, tmp)
        irows.append(ik)
    ti = jnp.concatenate(irows, axis=0)
    tw = jnp.concatenate(wrows, axis=0)
    tw = tw / (jnp.sum(tw, axis=0, keepdims=True) + 1e-20) * SCALE
    idx_ref[...] = ti
    w_ref[...] = tw


def _expert_kernel(rt_ref, rw_ref, nck_ref, bt_ref, xs_ref, gw_ref, uw_ref,
                   dw_ref, out_ref, xt_ref, yt_ref):
    e = pl.program_id(0)

    @pl.when(e == 0)
    def _():
        out_ref[...] = jnp.zeros_like(out_ref)

    def _chunk(ci, carry):
        tile = bt_ref[e] + ci
        for mi in range(_M):
            src = pl.multiple_of(rt_ref[tile, mi], _P)
            xt_ref[mi:mi + _P * _ST:_ST, :] = xs_ref[pl.ds(src, _P), :]
        x = jnp.concatenate(
            [xt_ref[pl.ds(k * _ST, _M), :] for k in range(_P)], axis=-1)
        g = lax.dot_general(x, gw_ref[0], (((1,), (1,)), ((), ())),
                            preferred_element_type=jnp.float32)
        u = lax.dot_general(x, uw_ref[0], (((1,), (1,)), ((), ())),
                            preferred_element_type=jnp.float32)
        h = g * jax.nn.sigmoid(g) * u
        y = lax.dot_general(h, dw_ref[0], (((1,), (1,)), ((), ())),
                            preferred_element_type=jnp.float32)
        for k in range(_P):
            yt_ref[pl.ds(k * _ST, _M), :] = y[:, k * 128:(k + 1) * 128]
        unroll = 4
        for base in range(0, _M, unroll):
            updated = []
            for q in range(unroll):
                mi = base + q
                dst = pl.multiple_of(rt_ref[tile, mi], _P)
                w = rw_ref[tile, mi]
                updated.append(out_ref[pl.ds(dst, _P), :]
                               + w * yt_ref[mi:mi + _P * _ST:_ST, :])
            for q in range(unroll):
                mi = base + q
                dst = pl.multiple_of(rt_ref[tile, mi], _P)
                out_ref[pl.ds(dst, _P), :] = updated[q]
        return carry

    lax.fori_loop(0, nck_ref[e], _chunk, 0)


def _scatter_kernel(dst_ref, fw_ref, ps_ref, pe_ref, ot_ref, ow_ref):
    def put(o, carry):
        for q in range(8):
            i = o * 8 + q
            d = dst_ref[i]
            ot_ref[d] = (i & (T - 1)) * _P
            ow_ref[d] = fw_ref[i]
        return carry

    lax.fori_loop(0, T * TOP_K // 8, put, 0)

    def pad_expert(e, carry):
        def fill(p, c2):
            ot_ref[p] = 0
            ow_ref[p] = 0.0
            return c2
        return lax.fori_loop(ps_ref[e], pe_ref[e], fill, carry)

    lax.fori_loop(0, E, pad_expert, 0)


def _combine_kernel(x_ref, sg_ref, su_ref, sd_ref, p_ref, o_ref):
    xb = x_ref[...]
    g = lax.dot_general(xb, sg_ref[...], (((1,), (1,)), ((), ())),
                        preferred_element_type=jnp.float32)
    u = lax.dot_general(xb, su_ref[...], (((1,), (1,)), ((), ())),
                        preferred_element_type=jnp.float32)
    h = g * jax.nn.sigmoid(g) * u
    sh = lax.dot_general(h, sd_ref[...], (((1,), (1,)), ((), ())),
                         preferred_element_type=jnp.float32)
    o_ref[...] = sh + p_ref[...]


def kernel(hidden_states, router_weight, router_bias, gate_w, up_w, down_w,
           shared_gate_w, shared_up_w, shared_down_w):
    tokens = hidden_states.reshape(T, H)
    nb = T // _TB // 2

    ti, tw = pl.pallas_call(
        _routing_kernel,
        grid=(T // _TB,),
        in_specs=[
            pl.BlockSpec((_TB, H), lambda b: (b, 0)),
            pl.BlockSpec((E, H), lambda b: (0, 0)),
            pl.BlockSpec((E, 1), lambda b: (0, 0)),
        ],
        out_specs=[
            pl.BlockSpec((TOP_K, _TB), lambda b: (0, b)),
            pl.BlockSpec((TOP_K, _TB), lambda b: (0, b)),
        ],
        out_shape=[
            jax.ShapeDtypeStruct((TOP_K, T), jnp.int32),
            jax.ShapeDtypeStruct((TOP_K, T), jnp.float32),
        ],
        compiler_params=pltpu.CompilerParams(
            dimension_semantics=("arbitrary",)),
        name="moe_routing",
    )(tokens, router_weight, router_bias.reshape(E, 1))

    flat_e = ti.reshape(-1)
    flat_w = tw.reshape(-1)
    nblk, blk = 64, T * TOP_K // 64
    oh = (flat_e[:, None] == jnp.arange(E, dtype=jnp.int32)[None, :]
          ).astype(jnp.float32)
    ohb = oh.reshape(nblk, blk, E)
    tril_b = jnp.tril(jnp.ones((blk, blk), jnp.float32))
    within = jnp.einsum('ij,bjk->bik', tril_b, ohb)
    blkc = jnp.sum(ohb, axis=1)
    tril_n = jnp.tril(jnp.ones((nblk, nblk), jnp.float32), k=-1)
    base = tril_n @ blkc
    counts = jnp.sum(blkc, axis=0).astype(jnp.int32)
    padded = ((counts + _M - 1) // _M) * _M
    pend = jnp.cumsum(padded)
    pstart = pend - padded
    nck = (padded // _M).astype(jnp.int32)
    btile = (pstart // _M).astype(jnp.int32)
    slot = within + (base[:, None, :] + pstart[None, None, :].astype(
        jnp.float32))
    dst = (jnp.sum(slot.reshape(T * TOP_K, E) * oh, axis=1)
           ).astype(jnp.int32) - 1
    rt_flat, rw_flat = pl.pallas_call(
        _scatter_kernel,
        in_specs=[
            pl.BlockSpec(memory_space=pltpu.SMEM),
            pl.BlockSpec(memory_space=pltpu.SMEM),
            pl.BlockSpec(memory_space=pltpu.SMEM),
            pl.BlockSpec(memory_space=pltpu.SMEM),
        ],
        out_specs=[
            pl.BlockSpec(memory_space=pltpu.SMEM),
            pl.BlockSpec(memory_space=pltpu.SMEM),
        ],
        out_shape=[
            jax.ShapeDtypeStruct((_NT * _M,), jnp.int32),
            jax.ShapeDtypeStruct((_NT * _M,), jnp.float32),
        ],
        name="moe_table_scatter",
    )(dst, flat_w, pstart + counts, pend)
    rt8 = rt_flat.reshape(_NT, _M)
    rw_tab = rw_flat.reshape(_NT, _M)

    xs = tokens.reshape(T * _P, 128)
    partial = pl.pallas_call(
        _expert_kernel,
        grid_spec=pltpu.PrefetchScalarGridSpec(
            num_scalar_prefetch=4,
            grid=(E,),
            in_specs=[
                pl.BlockSpec((T * _P, 128), lambda e, rt, rw, nc, bt: (0, 0)),
                pl.BlockSpec((1, I, H), lambda e, rt, rw, nc, bt: (e, 0, 0)),
                pl.BlockSpec((1, I, H), lambda e, rt, rw, nc, bt: (e, 0, 0)),
                pl.BlockSpec((1, H, I), lambda e, rt, rw, nc, bt: (e, 0, 0)),
            ],
            out_specs=pl.BlockSpec((T * _P, 128),
                                   lambda e, rt, rw, nc, bt: (0, 0)),
            scratch_shapes=[
                pltpu.VMEM((_ST * _P, 128), jnp.float32),
                pltpu.VMEM((_ST * _P, 128), jnp.float32),
            ],
        ),
        out_shape=jax.ShapeDtypeStruct((T * _P, 128), jnp.float32),
        compiler_params=pltpu.CompilerParams(
            dimension_semantics=("arbitrary",),
            vmem_limit_bytes=50 * 1024 * 1024),
        name="moe_experts",
    )(rt8, rw_tab, nck, btile, xs, gate_w, up_w, down_w)

    p2 = partial.reshape(T, H)
    out = pl.pallas_call(
        _combine_kernel,
        grid=(T // _TB,),
        in_specs=[
            pl.BlockSpec((_TB, H), lambda b: (b, 0)),
            pl.BlockSpec((SH_I, H), lambda b: (0, 0)),
            pl.BlockSpec((SH_I, H), lambda b: (0, 0)),
            pl.BlockSpec((H, SH_I), lambda b: (0, 0)),
            pl.BlockSpec((_TB, H), lambda b: (b, 0)),
        ],
        out_specs=pl.BlockSpec((_TB, H), lambda b: (b, 0)),
        out_shape=jax.ShapeDtypeStruct((T, H), jnp.float32),
        compiler_params=pltpu.CompilerParams(
            dimension_semantics=("arbitrary",),
            vmem_limit_bytes=50 * 1024 * 1024),
        name="moe_shared_combine",
    )(tokens, shared_gate_w, shared_up_w, shared_down_w, p2)
    return out.reshape(B, S, H)
